# Initial kernel scaffold; baseline (speedup 1.0000x reference)
import jax
import jax.numpy as jnp
from jax.experimental import pallas as pl


def kernel(x, W1, b1, W2, b2):
    raise NotImplementedError("write your pallas kernel here")


# speedup vs baseline: 4.6738x; 4.6738x over previous
import jax
import jax.numpy as jnp
from jax.experimental import pallas as pl
from jax.experimental.pallas import tpu as pltpu

_DEPTH_SCALES = 4


def _gather_body(idx_ref, *refs):
    k = (len(refs) - 1)
    out_ref = refs[-1]
    for t in range(k):
        out_ref[0, t] = refs[t][0, 0]


def _gather_planes(x, idx_flat, k_per_step=8):
    n, c, h, w = x.shape
    grid = (n, c // k_per_step)

    def in_map(t):
        return lambda i, j, idx_ref: (i, idx_ref[i, k_per_step * j + t], 0, 0)

    return pl.pallas_call(
        _gather_body,
        grid_spec=pltpu.PrefetchScalarGridSpec(
            num_scalar_prefetch=1,
            grid=grid,
            in_specs=[pl.BlockSpec((1, 1, h, w), in_map(t))
                      for t in range(k_per_step)],
            out_specs=pl.BlockSpec((1, k_per_step, h, w),
                                   lambda i, j, idx_ref: (i, j, 0, 0)),
        ),
        out_shape=jax.ShapeDtypeStruct((n, c, h, w), x.dtype),
    )(idx_flat, *([x] * k_per_step))


def kernel(x, W1, b1, W2, b2):
    n, c, h, w = x.shape
    d = _DEPTH_SCALES
    block_size = c // d
    avg = jnp.mean(x, axis=(2, 3))
    mx = jnp.max(x, axis=(2, 3))

    def mlp(v):
        hdn = jnp.maximum(v @ W1 + b1, 0.0)
        return hdn @ W2 + b2

    attn = jax.nn.sigmoid(mlp(avg) + mlp(mx))
    attn = attn.reshape(n, c, d)
    attn_t = jnp.transpose(attn, (0, 2, 1))
    _, idx = jax.lax.top_k(attn_t, block_size)
    idx_flat = idx.reshape(n, d * block_size).astype(jnp.int32)
    return _gather_planes(x, idx_flat)


# speedup vs baseline: 7.2285x; 1.5466x over previous
import jax
import jax.numpy as jnp
from jax.experimental import pallas as pl
from jax.experimental.pallas import tpu as pltpu

_DEPTH_SCALES = 4


def _gather_body(idx_ref, *refs):
    k = (len(refs) - 1)
    out_ref = refs[-1]
    for t in range(k):
        out_ref[0, t] = refs[t][0, 0]


def _gather_planes(x, idx_flat, k_per_step=32):
    n, c, h, w = x.shape
    grid = (n, c // k_per_step)

    def in_map(t):
        return lambda i, j, idx_ref: (i, idx_ref[i, k_per_step * j + t], 0, 0)

    return pl.pallas_call(
        _gather_body,
        grid_spec=pltpu.PrefetchScalarGridSpec(
            num_scalar_prefetch=1,
            grid=grid,
            in_specs=[pl.BlockSpec((1, 1, h, w), in_map(t))
                      for t in range(k_per_step)],
            out_specs=pl.BlockSpec((1, k_per_step, h, w),
                                   lambda i, j, idx_ref: (i, j, 0, 0)),
        ),
        out_shape=jax.ShapeDtypeStruct((n, c, h, w), x.dtype),
    )(idx_flat, *([x] * k_per_step))


def kernel(x, W1, b1, W2, b2):
    n, c, h, w = x.shape
    d = _DEPTH_SCALES
    block_size = c // d
    avg = jnp.mean(x, axis=(2, 3))
    mx = jnp.max(x, axis=(2, 3))

    def mlp(v):
        hdn = jnp.maximum(v @ W1 + b1, 0.0)
        return hdn @ W2 + b2

    attn = jax.nn.sigmoid(mlp(avg) + mlp(mx))
    attn = attn.reshape(n, c, d)
    attn_t = jnp.transpose(attn, (0, 2, 1))
    _, idx = jax.lax.top_k(attn_t, block_size)
    idx_flat = idx.reshape(n, d * block_size).astype(jnp.int32)
    return _gather_planes(x, idx_flat)


# speedup vs baseline: 7.6909x; 1.0640x over previous
import jax
import jax.numpy as jnp
from jax.experimental import pallas as pl
from jax.experimental.pallas import tpu as pltpu

_DEPTH_SCALES = 4


def _gather_body(idx_ref, *refs):
    k = (len(refs) - 1)
    out_ref = refs[-1]
    for t in range(k):
        out_ref[0, t] = refs[t][0, 0]


def _gather_planes(x, idx_flat, k_per_step=96):
    n, c, h, w = x.shape
    grid = (n, c // k_per_step)

    def in_map(t):
        return lambda i, j, idx_ref: (i, idx_ref[i, k_per_step * j + t], 0, 0)

    return pl.pallas_call(
        _gather_body,
        grid_spec=pltpu.PrefetchScalarGridSpec(
            num_scalar_prefetch=1,
            grid=grid,
            in_specs=[pl.BlockSpec((1, 1, h, w), in_map(t))
                      for t in range(k_per_step)],
            out_specs=pl.BlockSpec((1, k_per_step, h, w),
                                   lambda i, j, idx_ref: (i, j, 0, 0)),
        ),
        out_shape=jax.ShapeDtypeStruct((n, c, h, w), x.dtype),
    )(idx_flat, *([x] * k_per_step))


def kernel(x, W1, b1, W2, b2):
    n, c, h, w = x.shape
    d = _DEPTH_SCALES
    block_size = c // d
    avg = jnp.mean(x, axis=(2, 3))
    mx = jnp.max(x, axis=(2, 3))

    def mlp(v):
        hdn = jnp.maximum(v @ W1 + b1, 0.0)
        return hdn @ W2 + b2

    attn = jax.nn.sigmoid(mlp(avg) + mlp(mx))
    attn = attn.reshape(n, c, d)
    attn_t = jnp.transpose(attn, (0, 2, 1))
    _, idx = jax.lax.top_k(attn_t, block_size)
    idx_flat = idx.reshape(n, d * block_size).astype(jnp.int32)
    return _gather_planes(x, idx_flat)
